# baseline (speedup 1.0000x reference)
import functools

import jax
import jax.numpy as jnp
from jax.experimental import pallas as pl
from jax.experimental.pallas import tpu as pltpu
from jax.experimental.pallas import tpu_sc as plsc

NUM_CLASSES = 100
EOS_COEF = 0.1
PAD_IDX = 1
SMOOTHING = 0.7
_CCR11 = [0.0, 0.0, 0.193425917, 0.412129084, 0.188929963, 0.0781296833,
          0.0509541413, 0.0312718553, 0.018483365, 0.0083924468, 0.00659406534]

_V = 10000
_ROWS = 64 * 30
_BR = 128
_GRID = _ROWS // _BR
_NL = 64 * 100
_RPAD = 2048
_TB = _RPAD // 32
_EPS = SMOOTHING / (_V - 2)


def _sc_gather_body(table_ref, rows_ref, out_ref, idx_v, rows_v, sem):
    wid = jax.lax.axis_index("s") * 2 + jax.lax.axis_index("c")
    base = wid * _TB
    pltpu.sync_copy(rows_ref.at[pl.ds(base, _TB)], idx_v)
    pltpu.async_copy(table_ref.at[idx_v], rows_v, sem).wait()
    pltpu.sync_copy(rows_v, out_ref.at[pl.ds(base, _TB)])


def _sc_gather(table128, rows):
    f = pl.kernel(
        _sc_gather_body,
        out_type=jax.ShapeDtypeStruct((_RPAD, 128), jnp.float32),
        mesh=plsc.VectorSubcoreMesh(core_axis_name="c", subcore_axis_name="s"),
        scratch_types=[
            pltpu.VMEM((_TB,), jnp.int32),
            pltpu.VMEM((_TB, 128), jnp.float32),
            pltpu.SemaphoreType.DMA,
        ],
    )
    return f(table128, rows)


def _loss_kernel(cap_ref, tcap_ref, g16_ref, lane_ref, tpad_ref,
                 logit_ref, tcls_ref, pc_ref, ct_ref, out_ref):
    i = pl.program_id(0)
    c_row = SMOOTHING * jnp.log(_EPS) + (1.0 - SMOOTHING) * jnp.log(1.0 - SMOOTHING)

    @pl.when(i == 0)
    def _small_losses():
        x = logit_ref[...]
        tc = tcls_ref[...]
        cid = jax.lax.broadcasted_iota(jnp.int32, x.shape, 1)
        m = jnp.max(x, axis=1, keepdims=True)
        lse = jnp.log(jnp.sum(jnp.exp(x - m), axis=1, keepdims=True)) + m
        xt = jnp.sum(jnp.where(cid == tc, x, 0.0), axis=1, keepdims=True)
        w = jnp.where(tc == NUM_CLASSES, EOS_COEF, 1.0)
        loss_ce = jnp.sum(w * (lse - xt), keepdims=True) / jnp.sum(w)

        pc = pc_ref[...]
        ct = ct_ref[...]
        j = jax.lax.broadcasted_iota(jnp.int32, pc.shape, 1)
        onehot = (j == ct)
        diff = (j - ct).astype(jnp.float32)
        gmask = jnp.exp(-diff * diff / 8.0)
        tgt = onehot.astype(jnp.float32)
        bce = (jnp.maximum(pc, 0.0) - pc * tgt
               + jnp.log1p(jnp.exp(-jnp.abs(pc))))
        coef = jnp.where(onehot, 1.0, 1.0 - gmask)
        wccr = jnp.zeros(pc.shape, jnp.float32)
        for k, v in enumerate(_CCR11):
            wccr = jnp.where(j == k, 1.0 - v, wccr)
        loss_counter = (jnp.sum(bce * wccr * coef, keepdims=True)
                        / (pc.shape[0] * pc.shape[1]))

        g16 = g16_ref[...]
        lane = lane_ref[...]
        tp = tpad_ref[...]
        li = jax.lax.broadcasted_iota(jnp.int32, g16.shape, 1)
        gv = jnp.sum(jnp.where(li == lane, g16, 0.0), axis=1, keepdims=True)
        lg = jnp.log(gv)
        term_g = jnp.where(tp == PAD_IDX, 0.0,
                           c_row + (_EPS - (1.0 - SMOOTHING)) * lg)
        out_ref[...] = (loss_ce + loss_counter
                        + jnp.sum(term_g, keepdims=True))

    x = cap_ref[...]
    t = tcap_ref[...]
    a = x[:, 0:4992] * x[:, 4992:9984]
    b = a[:, 0:2432] * a[:, 2432:4864]
    c = b[:, 0:1152] * b[:, 1152:2304]
    s_all = (jnp.sum(jnp.log(c), axis=1, keepdims=True)
             + jnp.sum(jnp.log(b[:, 2304:2432]), axis=1, keepdims=True)
             + jnp.sum(jnp.log(a[:, 4864:4992]), axis=1, keepdims=True)
             + jnp.sum(jnp.log(x[:, 9984:10000]), axis=1, keepdims=True))
    lp1 = jnp.log(x[:, PAD_IDX:PAD_IDX + 1])
    kl = jnp.where(t == PAD_IDX, 0.0, -_EPS * (s_all - lp1))
    out_ref[...] += jnp.sum(kl, keepdims=True)


@jax.jit
def kernel(pred_logits, target_classes, pred_count, counter_target,
           pred_captions, target_caption):
    cap = pred_captions.reshape(_ROWS, _V)
    table128 = pred_captions.reshape(_ROWS * _V // 128, 128)
    t = target_caption.reshape(_ROWS).astype(jnp.int32)
    flat = jnp.arange(_ROWS, dtype=jnp.int32) * _V + t
    rows = jnp.concatenate([flat // 128, jnp.zeros(_RPAD - _ROWS, jnp.int32)])
    lane = jnp.concatenate([flat % 128, jnp.zeros(_RPAD - _ROWS, jnp.int32)])
    tpad = jnp.concatenate(
        [t, jnp.full(_RPAD - _ROWS, PAD_IDX, jnp.int32)]).reshape(_RPAD, 1)

    g16 = _sc_gather(table128, rows)

    logits = pred_logits.reshape(_NL, NUM_CLASSES + 1)
    tcls = target_classes.reshape(_NL, 1).astype(jnp.int32)
    ct = counter_target.reshape(64, 1).astype(jnp.int32)

    out = pl.pallas_call(
        _loss_kernel,
        grid=(_GRID,),
        in_specs=[
            pl.BlockSpec((_BR, _V), lambda i: (i, 0)),
            pl.BlockSpec((_BR, 1), lambda i: (i, 0)),
            pl.BlockSpec((_RPAD, 128), lambda i: (0, 0)),
            pl.BlockSpec((_RPAD, 1), lambda i: (0, 0)),
            pl.BlockSpec((_RPAD, 1), lambda i: (0, 0)),
            pl.BlockSpec((_NL, NUM_CLASSES + 1), lambda i: (0, 0)),
            pl.BlockSpec((_NL, 1), lambda i: (0, 0)),
            pl.BlockSpec((64, 11), lambda i: (0, 0)),
            pl.BlockSpec((64, 1), lambda i: (0, 0)),
        ],
        out_specs=pl.BlockSpec((1, 1), lambda i: (0, 0)),
        out_shape=jax.ShapeDtypeStruct((1, 1), jnp.float32),
    )(cap, t.reshape(_ROWS, 1), g16, lane.reshape(_RPAD, 1), tpad,
      logits, tcls, pred_count, ct)
    return out[0, 0]


# speedup vs baseline: 8.2596x; 8.2596x over previous
import functools

import jax
import jax.numpy as jnp
from jax.experimental import pallas as pl
from jax.experimental.pallas import tpu as pltpu
from jax.experimental.pallas import tpu_sc as plsc

NUM_CLASSES = 100
EOS_COEF = 0.1
PAD_IDX = 1
SMOOTHING = 0.7
_CCR11 = [0.0, 0.0, 0.193425917, 0.412129084, 0.188929963, 0.0781296833,
          0.0509541413, 0.0312718553, 0.018483365, 0.0083924468, 0.00659406534]

_V = 10000
_ROWS = 64 * 30
_BR = 128
_GRID = _ROWS // _BR
_NL = 64 * 100
_RPAD = 2048
_TB = _RPAD // 32
_EPS = SMOOTHING / (_V - 2)


def _sc_gather_body(table_ref, rows_ref, out_ref, idx_v, rows_v, sem):
    wid = jax.lax.axis_index("s") * 2 + jax.lax.axis_index("c")
    base = wid * _TB
    pltpu.sync_copy(rows_ref.at[pl.ds(base, _TB)], idx_v)
    pltpu.async_copy(table_ref.at[idx_v], rows_v, sem).wait()
    pltpu.sync_copy(rows_v, out_ref.at[pl.ds(base, _TB)])


def _sc_gather(table128, rows):
    f = pl.kernel(
        _sc_gather_body,
        out_type=jax.ShapeDtypeStruct((_RPAD, 128), jnp.float32),
        mesh=plsc.VectorSubcoreMesh(core_axis_name="c", subcore_axis_name="s"),
        scratch_types=[
            pltpu.VMEM((_TB,), jnp.int32),
            pltpu.VMEM((_TB, 128), jnp.float32),
            pltpu.SemaphoreType.DMA,
        ],
    )
    return f(table128, rows)


def _loss_kernel(cap_ref, tcap_ref,
                 logit_ref, tcls_ref, pc_ref, ct_ref, out_ref):
    i = pl.program_id(0)

    @pl.when(i == 0)
    def _small_losses():
        x = logit_ref[...]
        tc = tcls_ref[...]
        cid = jax.lax.broadcasted_iota(jnp.int32, x.shape, 1)
        m = jnp.max(x, axis=1, keepdims=True)
        lse = jnp.log(jnp.sum(jnp.exp(x - m), axis=1, keepdims=True)) + m
        xt = jnp.sum(jnp.where(cid == tc, x, 0.0), axis=1, keepdims=True)
        w = jnp.where(tc == NUM_CLASSES, EOS_COEF, 1.0)
        loss_ce = jnp.sum(w * (lse - xt), keepdims=True) / jnp.sum(w)

        pc = pc_ref[...]
        ct = ct_ref[...]
        j = jax.lax.broadcasted_iota(jnp.int32, pc.shape, 1)
        onehot = (j == ct)
        diff = (j - ct).astype(jnp.float32)
        gmask = jnp.exp(-diff * diff / 8.0)
        tgt = onehot.astype(jnp.float32)
        bce = (jnp.maximum(pc, 0.0) - pc * tgt
               + jnp.log1p(jnp.exp(-jnp.abs(pc))))
        coef = jnp.where(onehot, 1.0, 1.0 - gmask)
        wccr = jnp.zeros(pc.shape, jnp.float32)
        for k, v in enumerate(_CCR11):
            wccr = jnp.where(j == k, 1.0 - v, wccr)
        loss_counter = (jnp.sum(bce * wccr * coef, keepdims=True)
                        / (pc.shape[0] * pc.shape[1]))

        out_ref[...] = loss_ce + loss_counter

    x = cap_ref[...]
    t = tcap_ref[...]
    vid = jax.lax.broadcasted_iota(jnp.int32, x.shape, 1)
    gv = jnp.sum(jnp.where(vid == t, x, 0.0), axis=1, keepdims=True)
    g = jnp.log(gv)
    a = x[:, 0:4992] * x[:, 4992:9984]
    b = a[:, 0:2432] * a[:, 2432:4864]
    c = b[:, 0:1152] * b[:, 1152:2304]
    s_all = (jnp.sum(jnp.log(c), axis=1, keepdims=True)
             + jnp.sum(jnp.log(b[:, 2304:2432]), axis=1, keepdims=True)
             + jnp.sum(jnp.log(a[:, 4864:4992]), axis=1, keepdims=True)
             + jnp.sum(jnp.log(x[:, 9984:10000]), axis=1, keepdims=True))
    c_row = SMOOTHING * jnp.log(_EPS) + (1.0 - SMOOTHING) * jnp.log(1.0 - SMOOTHING)
    lp1 = jnp.log(x[:, PAD_IDX:PAD_IDX + 1])
    kl = jnp.where(t == PAD_IDX, 0.0,
                   c_row - _EPS * (s_all - lp1)
                   + (_EPS - (1.0 - SMOOTHING)) * g)
    out_ref[...] += jnp.sum(kl, keepdims=True)


@jax.jit
def kernel(pred_logits, target_classes, pred_count, counter_target,
           pred_captions, target_caption):
    cap = pred_captions.reshape(_ROWS, _V)
    t = target_caption.reshape(_ROWS).astype(jnp.int32)
    logits = pred_logits.reshape(_NL, NUM_CLASSES + 1)
    tcls = target_classes.reshape(_NL, 1).astype(jnp.int32)
    ct = counter_target.reshape(64, 1).astype(jnp.int32)

    out = pl.pallas_call(
        _loss_kernel,
        grid=(_GRID,),
        in_specs=[
            pl.BlockSpec((_BR, _V), lambda i: (i, 0)),
            pl.BlockSpec((_BR, 1), lambda i: (i, 0)),
            pl.BlockSpec((_NL, NUM_CLASSES + 1), lambda i: (0, 0)),
            pl.BlockSpec((_NL, 1), lambda i: (0, 0)),
            pl.BlockSpec((64, 11), lambda i: (0, 0)),
            pl.BlockSpec((64, 1), lambda i: (0, 0)),
        ],
        out_specs=pl.BlockSpec((1, 1), lambda i: (0, 0)),
        out_shape=jax.ShapeDtypeStruct((1, 1), jnp.float32),
    )(cap, t.reshape(_ROWS, 1), logits, tcls, pred_count, ct)
    return out[0, 0]


# speedup vs baseline: 14.1259x; 1.7102x over previous
import jax
import jax.numpy as jnp
from jax.experimental import pallas as pl

NUM_CLASSES = 100
EOS_COEF = 0.1
PAD_IDX = 1
SMOOTHING = 0.7
_CCR11 = [0.0, 0.0, 0.193425917, 0.412129084, 0.188929963, 0.0781296833,
          0.0509541413, 0.0312718553, 0.018483365, 0.0083924468, 0.00659406534]

_V = 10000
_B = 64
_S = 30
_BB = 8
_GRID = _B // _BB
_EPS = SMOOTHING / (_V - 2)


def _loss_kernel(cap_ref, tcap_ref, logit_ref, tcls_ref, pc_ref, ct_ref,
                 out_ref):
    i = pl.program_id(0)

    @pl.when(i == 0)
    def _small_losses():
        x = logit_ref[...]
        tc = tcls_ref[...]
        cid = jax.lax.broadcasted_iota(jnp.int32, x.shape, 2)
        m = jnp.max(x, axis=2, keepdims=True)
        lse = jnp.log(jnp.sum(jnp.exp(x - m), axis=2, keepdims=True)) + m
        xt = jnp.sum(jnp.where(cid == tc, x, 0.0), axis=2, keepdims=True)
        w = jnp.where(tc == NUM_CLASSES, EOS_COEF, 1.0)
        loss_ce = jnp.sum(w * (lse - xt), keepdims=True) / jnp.sum(w)

        pc = pc_ref[...]
        ct = ct_ref[...]
        j = jax.lax.broadcasted_iota(jnp.int32, pc.shape, 2)
        onehot = (j == ct)
        diff = (j - ct).astype(jnp.float32)
        gmask = jnp.exp(-diff * diff / 8.0)
        tgt = onehot.astype(jnp.float32)
        bce = (jnp.maximum(pc, 0.0) - pc * tgt
               + jnp.log1p(jnp.exp(-jnp.abs(pc))))
        coef = jnp.where(onehot, 1.0, 1.0 - gmask)
        wccr = jnp.zeros(pc.shape, jnp.float32)
        for k, v in enumerate(_CCR11):
            wccr = jnp.where(j == k, 1.0 - v, wccr)
        loss_counter = jnp.sum(bce * wccr * coef, keepdims=True) / (64 * 11)

        out_ref[...] = loss_ce + loss_counter

    x = cap_ref[...]
    t = tcap_ref[...]
    vid = jax.lax.broadcasted_iota(jnp.int32, x.shape, 2)
    gv = jnp.sum(jnp.where(vid == t, x, 0.0), axis=2, keepdims=True)
    g = jnp.log(gv)
    a = x[:, :, 0:4992] * x[:, :, 4992:9984]
    b = a[:, :, 0:2432] * a[:, :, 2432:4864]
    c = b[:, :, 0:1152] * b[:, :, 1152:2304]
    s_all = (jnp.sum(jnp.log(c), axis=2, keepdims=True)
             + jnp.sum(jnp.log(b[:, :, 2304:2432]), axis=2, keepdims=True)
             + jnp.sum(jnp.log(a[:, :, 4864:4992]), axis=2, keepdims=True)
             + jnp.sum(jnp.log(x[:, :, 9984:10000]), axis=2, keepdims=True))
    c_row = (SMOOTHING * jnp.log(_EPS)
             + (1.0 - SMOOTHING) * jnp.log(1.0 - SMOOTHING))
    lp1 = jnp.log(x[:, :, PAD_IDX:PAD_IDX + 1])
    kl = jnp.where(t == PAD_IDX, 0.0,
                   c_row - _EPS * (s_all - lp1)
                   + (_EPS - (1.0 - SMOOTHING)) * g)
    out_ref[...] += jnp.sum(kl, keepdims=True)


@jax.jit
def kernel(pred_logits, target_classes, pred_count, counter_target,
           pred_captions, target_caption):
    tcap = target_caption.astype(jnp.int32).reshape(_B, _S, 1)
    tcls = target_classes.astype(jnp.int32).reshape(_B, 100, 1)
    pc = pred_count.reshape(1, _B, 11)
    ct = counter_target.astype(jnp.int32).reshape(1, _B, 1)

    out = pl.pallas_call(
        _loss_kernel,
        grid=(_GRID,),
        in_specs=[
            pl.BlockSpec((_BB, _S, _V), lambda i: (i, 0, 0)),
            pl.BlockSpec((_BB, _S, 1), lambda i: (i, 0, 0)),
            pl.BlockSpec((_B, 100, NUM_CLASSES + 1), lambda i: (0, 0, 0)),
            pl.BlockSpec((_B, 100, 1), lambda i: (0, 0, 0)),
            pl.BlockSpec((1, _B, 11), lambda i: (0, 0, 0)),
            pl.BlockSpec((1, _B, 1), lambda i: (0, 0, 0)),
        ],
        out_specs=pl.BlockSpec((1, 1, 1), lambda i: (0, 0, 0)),
        out_shape=jax.ShapeDtypeStruct((1, 1, 1), jnp.float32),
    )(pred_captions, tcap, pred_logits, tcls, pc, ct)
    return out[0, 0, 0]
